# Initial kernel scaffold; baseline (speedup 1.0000x reference)
import jax
import jax.numpy as jnp
from jax.experimental import pallas as pl


def kernel(z_e, emb):
    raise NotImplementedError("write your pallas kernel here")


# speedup vs baseline: 1.0217x; 1.0217x over previous
import functools

import jax
import jax.numpy as jnp
from jax import lax
from jax.experimental import pallas as pl
from jax.experimental.pallas import tpu as pltpu
from jax.experimental.pallas import tpu_sc as plsc

_BETA = 0.25
_BM = 512
_BN = 2048
_INT_MAX = 2**31 - 1


def _argmin_body(zsq_ref, esq_ref, z_ref, emb_ref, idx_ref, m_scr, a_scr):
    n = pl.program_id(1)
    mm = lax.dot_general(
        z_ref[...], emb_ref[...],
        dimension_numbers=(((1,), (1,)), ((), ())),
        preferred_element_type=jnp.float32,
    )
    dist = (zsq_ref[...] - 2.0 * mm) + esq_ref[...]
    mloc = jnp.min(dist, axis=1, keepdims=True)
    iota = lax.broadcasted_iota(jnp.int32, dist.shape, 1) + n * _BN
    aloc = jnp.min(jnp.where(dist == mloc, iota, _INT_MAX),
                   axis=1, keepdims=True)

    @pl.when(n == 0)
    def _():
        m_scr[...] = mloc
        a_scr[...] = aloc

    @pl.when(n > 0)
    def _():
        better = mloc < m_scr[...]
        a_scr[...] = jnp.where(better, aloc, a_scr[...])
        m_scr[...] = jnp.where(better, mloc, m_scr[...])

    @pl.when(n == pl.num_programs(1) - 1)
    def _():
        idx_ref[...] = a_scr[...]


def _argmin_call(z, emb, zsq, esq):
    m, k = z.shape
    n = emb.shape[0]
    grid = (m // _BM, n // _BN)
    return pl.pallas_call(
        _argmin_body,
        grid=grid,
        in_specs=[
            pl.BlockSpec((_BM, 1), lambda i, j: (i, 0)),
            pl.BlockSpec((1, _BN), lambda i, j: (0, j)),
            pl.BlockSpec((_BM, k), lambda i, j: (i, 0)),
            pl.BlockSpec((_BN, k), lambda i, j: (j, 0)),
        ],
        out_specs=pl.BlockSpec((_BM, 1), lambda i, j: (i, 0)),
        out_shape=jax.ShapeDtypeStruct((m, 1), jnp.int32),
        scratch_shapes=[
            pltpu.VMEM((_BM, 1), jnp.float32),
            pltpu.VMEM((_BM, 1), jnp.int32),
        ],
        compiler_params=pltpu.CompilerParams(
            dimension_semantics=("arbitrary", "arbitrary"),
        ),
    )(zsq, esq, z, emb)


def _make_sc_gather(v, d, b):
    info = plsc.get_sparse_core_info()
    nw = info.num_cores * info.num_subcores
    b_per_w = b // nw
    mesh = plsc.VectorSubcoreMesh(core_axis_name="c", subcore_axis_name="s")

    @functools.partial(
        pl.kernel, mesh=mesh,
        out_type=jax.ShapeDtypeStruct((b, d), jnp.float32),
        scratch_types=[
            pltpu.VMEM((b_per_w,), jnp.int32),
            pltpu.VMEM((b_per_w, d), jnp.float32),
            pltpu.SemaphoreType.DMA,
        ],
    )
    def gather(table_hbm, idx_hbm, out_hbm, idx_v, rows_v, sem):
        wid = lax.axis_index("s") * info.num_cores + lax.axis_index("c")
        base = wid * b_per_w
        pltpu.sync_copy(idx_hbm.at[pl.ds(base, b_per_w)], idx_v)
        pltpu.async_copy(table_hbm.at[idx_v], rows_v, sem).wait()
        pltpu.sync_copy(rows_v, out_hbm.at[pl.ds(base, b_per_w)])

    return gather


def _st_loss_body(z_ref, zq_ref, st_ref, loss_ref, acc):
    i = pl.program_id(0)
    z = z_ref[...]
    q = zq_ref[...]
    t = q - z
    st_ref[...] = z + t
    part = jnp.sum(t * t)

    @pl.when(i == 0)
    def _():
        acc[0] = part

    @pl.when(i > 0)
    def _():
        acc[0] = acc[0] + part

    @pl.when(i == pl.num_programs(0) - 1)
    def _():
        loss_ref[0, 0] = acc[0]


def _st_loss_call(z, zq):
    m, k = z.shape
    return pl.pallas_call(
        _st_loss_body,
        grid=(m // _BM,),
        in_specs=[
            pl.BlockSpec((_BM, k), lambda i: (i, 0)),
            pl.BlockSpec((_BM, k), lambda i: (i, 0)),
        ],
        out_specs=[
            pl.BlockSpec((_BM, k), lambda i: (i, 0)),
            pl.BlockSpec(memory_space=pltpu.SMEM),
        ],
        out_shape=[
            jax.ShapeDtypeStruct((m, k), jnp.float32),
            jax.ShapeDtypeStruct((1, 1), jnp.float32),
        ],
        scratch_shapes=[pltpu.SMEM((1,), jnp.float32)],
    )(z, zq)


def kernel(z_e, emb):
    b, c, h, w = z_e.shape
    n_codes = emb.shape[0]
    z = jnp.transpose(z_e, (0, 2, 3, 1)).reshape(-1, c)
    zsq = jnp.sum(z * z, axis=1, keepdims=True)
    esq = jnp.sum(emb * emb, axis=1)[None, :]

    idx2 = _argmin_call(z, emb, zsq, esq)
    idx = idx2.reshape(-1)

    zq = _make_sc_gather(n_codes, c, z.shape[0])(emb, idx)

    st_flat, loss_sum = _st_loss_call(z, zq)
    mean_sq = loss_sum[0, 0] / jnp.float32(z.size)
    loss = mean_sq + jnp.float32(_BETA) * mean_sq

    z_q_st = jnp.transpose(st_flat.reshape(b, h, w, c), (0, 3, 1, 2))
    return (z_q_st, loss, idx.reshape(b, h, w))


# speedup vs baseline: 1.1435x; 1.1192x over previous
import functools

import jax
import jax.numpy as jnp
from jax import lax
from jax.experimental import pallas as pl
from jax.experimental.pallas import tpu as pltpu
from jax.experimental.pallas import tpu_sc as plsc

_BETA = 0.25
_BM = 512
_BN = 2048
_INT_MAX = 2**31 - 1


def _argmin_body(zsq_ref, esq_ref, z_ref, emb_ref, idx_ref, loss_ref, acc):
    i = pl.program_id(0)
    n_chunks = emb_ref.shape[0] // _BN
    zm = z_ref[...] * -2.0
    zsq = zsq_ref[...]
    m_run = None
    a_run = None
    for c in range(n_chunks):
        mm = lax.dot_general(
            zm, emb_ref[pl.ds(c * _BN, _BN), :],
            dimension_numbers=(((1,), (1,)), ((), ())),
            preferred_element_type=jnp.float32,
        )
        dist = (zsq + mm) + esq_ref[:, pl.ds(c * _BN, _BN)]
        mloc = jnp.min(dist, axis=1, keepdims=True)
        iota = lax.broadcasted_iota(jnp.int32, dist.shape, 1)
        aloc = jnp.min(jnp.where(dist == mloc, iota, _INT_MAX),
                       axis=1, keepdims=True) + c * _BN
        if c == 0:
            m_run, a_run = mloc, aloc
        else:
            better = mloc < m_run
            a_run = jnp.where(better, aloc, a_run)
            m_run = jnp.where(better, mloc, m_run)

    idx_ref[...] = a_run
    part = jnp.sum(m_run)

    @pl.when(i == 0)
    def _():
        acc[0] = part

    @pl.when(i > 0)
    def _():
        acc[0] = acc[0] + part

    @pl.when(i == pl.num_programs(0) - 1)
    def _():
        loss_ref[0, 0] = acc[0]


def _argmin_call(z, emb, zsq, esq):
    m, k = z.shape
    n = emb.shape[0]
    return pl.pallas_call(
        _argmin_body,
        grid=(m // _BM,),
        in_specs=[
            pl.BlockSpec((_BM, 1), lambda i: (i, 0)),
            pl.BlockSpec((1, n), lambda i: (0, 0)),
            pl.BlockSpec((_BM, k), lambda i: (i, 0)),
            pl.BlockSpec((n, k), lambda i: (0, 0)),
        ],
        out_specs=[
            pl.BlockSpec((_BM, 1), lambda i: (i, 0)),
            pl.BlockSpec(memory_space=pltpu.SMEM),
        ],
        out_shape=[
            jax.ShapeDtypeStruct((m, 1), jnp.int32),
            jax.ShapeDtypeStruct((1, 1), jnp.float32),
        ],
        scratch_shapes=[pltpu.SMEM((1,), jnp.float32)],
        compiler_params=pltpu.CompilerParams(
            dimension_semantics=("arbitrary",),
        ),
    )(zsq, esq, z, emb)


def _make_sc_gather_st(v, d, b):
    info = plsc.get_sparse_core_info()
    nw = info.num_cores * info.num_subcores
    b_per_w = b // nw
    lanes = info.num_lanes
    mesh = plsc.VectorSubcoreMesh(core_axis_name="c", subcore_axis_name="s")

    @functools.partial(
        pl.kernel, mesh=mesh,
        out_type=jax.ShapeDtypeStruct((b, d), jnp.float32),
        scratch_types=[
            pltpu.VMEM((b_per_w,), jnp.int32),
            pltpu.VMEM((b_per_w, d), jnp.float32),
            pltpu.VMEM((b_per_w, d), jnp.float32),
            pltpu.SemaphoreType.DMA,
        ],
    )
    def gather_st(table_hbm, idx_hbm, z_hbm, out_hbm, idx_v, rows_v, z_v, sem):
        wid = lax.axis_index("s") * info.num_cores + lax.axis_index("c")
        base = wid * b_per_w
        pltpu.sync_copy(idx_hbm.at[pl.ds(base, b_per_w)], idx_v)
        cp = pltpu.async_copy(table_hbm.at[idx_v], rows_v, sem)
        pltpu.sync_copy(z_hbm.at[pl.ds(base, b_per_w)], z_v)
        cp.wait()

        def row_body(r, carry):
            for j in range(d // lanes):
                zv = z_v[r, pl.ds(j * lanes, lanes)]
                ev = rows_v[r, pl.ds(j * lanes, lanes)]
                rows_v[r, pl.ds(j * lanes, lanes)] = zv + (ev - zv)
            return carry

        lax.fori_loop(0, b_per_w, row_body, 0)
        pltpu.sync_copy(rows_v, out_hbm.at[pl.ds(base, b_per_w)])

    return gather_st


def kernel(z_e, emb):
    b, c, h, w = z_e.shape
    n_codes = emb.shape[0]
    z = jnp.transpose(z_e, (0, 2, 3, 1)).reshape(-1, c)
    zsq = jnp.sum(z * z, axis=1, keepdims=True)
    esq = jnp.sum(emb * emb, axis=1)[None, :]

    idx2, loss_sum = _argmin_call(z, emb, zsq, esq)
    idx = idx2.reshape(-1)

    st_flat = _make_sc_gather_st(n_codes, c, z.shape[0])(emb, idx, z)

    mean_sq = loss_sum[0, 0] / jnp.float32(z.size)
    loss = mean_sq + jnp.float32(_BETA) * mean_sq

    z_q_st = jnp.transpose(st_flat.reshape(b, h, w, c), (0, 3, 1, 2))
    return (z_q_st, loss, idx.reshape(b, h, w))


# speedup vs baseline: 1.2197x; 1.0666x over previous
import functools

import jax
import jax.numpy as jnp
from jax import lax
from jax.experimental import pallas as pl
from jax.experimental.pallas import tpu as pltpu
from jax.experimental.pallas import tpu_sc as plsc

_BETA = 0.25
_BM = 512
_BN = 2048
_INT_MAX = 2**31 - 1


def _argmin_body(zsq_ref, esq_ref, z_ref, emb_ref, idx_ref, loss_ref, acc):
    i = pl.program_id(0)
    n_chunks = emb_ref.shape[0] // _BN
    zm = z_ref[...] * -2.0
    zsq = zsq_ref[...]
    iota = lax.broadcasted_iota(jnp.int32, (_BM, _BN), 1).astype(jnp.float32)
    m_run = None
    a_run = None
    for c in range(n_chunks):
        mm = lax.dot_general(
            zm, emb_ref[pl.ds(c * _BN, _BN), :],
            dimension_numbers=(((1,), (1,)), ((), ())),
            preferred_element_type=jnp.float32,
        )
        dist = (zsq + mm) + esq_ref[:, pl.ds(c * _BN, _BN)]
        mloc = jnp.min(dist, axis=1, keepdims=True)
        aloc = jnp.min(jnp.where(dist == mloc, iota, jnp.inf),
                       axis=1, keepdims=True) + float(c * _BN)
        if c == 0:
            m_run, a_run = mloc, aloc
        else:
            better = mloc < m_run
            a_run = jnp.where(better, aloc, a_run)
            m_run = jnp.where(better, mloc, m_run)

    idx_ref[...] = a_run.astype(jnp.int32)
    part = jnp.sum(m_run)

    @pl.when(i == 0)
    def _():
        acc[0] = part

    @pl.when(i > 0)
    def _():
        acc[0] = acc[0] + part

    @pl.when(i == pl.num_programs(0) - 1)
    def _():
        loss_ref[0, 0] = acc[0]


def _argmin_call(z, emb, zsq, esq):
    m, k = z.shape
    n = emb.shape[0]
    return pl.pallas_call(
        _argmin_body,
        grid=(m // _BM,),
        in_specs=[
            pl.BlockSpec((_BM, 1), lambda i: (i, 0)),
            pl.BlockSpec((1, n), lambda i: (0, 0)),
            pl.BlockSpec((_BM, k), lambda i: (i, 0)),
            pl.BlockSpec((n, k), lambda i: (0, 0)),
        ],
        out_specs=[
            pl.BlockSpec((_BM, 1), lambda i: (i, 0)),
            pl.BlockSpec(memory_space=pltpu.SMEM),
        ],
        out_shape=[
            jax.ShapeDtypeStruct((m, 1), jnp.int32),
            jax.ShapeDtypeStruct((1, 1), jnp.float32),
        ],
        scratch_shapes=[pltpu.SMEM((1,), jnp.float32)],
        compiler_params=pltpu.CompilerParams(
            dimension_semantics=("arbitrary",),
        ),
    )(zsq, esq, z, emb)


def _make_sc_gather_st(v, d, b):
    info = plsc.get_sparse_core_info()
    nw = info.num_cores * info.num_subcores
    b_per_w = b // nw
    lanes = info.num_lanes
    mesh = plsc.VectorSubcoreMesh(core_axis_name="c", subcore_axis_name="s")

    @functools.partial(
        pl.kernel, mesh=mesh,
        out_type=jax.ShapeDtypeStruct((b, d), jnp.float32),
        scratch_types=[
            pltpu.VMEM((b_per_w,), jnp.int32),
            pltpu.VMEM((b_per_w, d), jnp.float32),
            pltpu.VMEM((b_per_w, d), jnp.float32),
            pltpu.SemaphoreType.DMA,
        ],
    )
    def gather_st(table_hbm, idx_hbm, z_hbm, out_hbm, idx_v, rows_v, z_v, sem):
        wid = lax.axis_index("s") * info.num_cores + lax.axis_index("c")
        base = wid * b_per_w
        pltpu.sync_copy(idx_hbm.at[pl.ds(base, b_per_w)], idx_v)
        cp = pltpu.async_copy(table_hbm.at[idx_v], rows_v, sem)
        pltpu.sync_copy(z_hbm.at[pl.ds(base, b_per_w)], z_v)
        cp.wait()

        def row_body(r, carry):
            for j in range(d // lanes):
                zv = z_v[r, pl.ds(j * lanes, lanes)]
                ev = rows_v[r, pl.ds(j * lanes, lanes)]
                rows_v[r, pl.ds(j * lanes, lanes)] = zv + (ev - zv)
            return carry

        lax.fori_loop(0, b_per_w, row_body, 0)
        pltpu.sync_copy(rows_v, out_hbm.at[pl.ds(base, b_per_w)])

    return gather_st


def kernel(z_e, emb):
    b, c, h, w = z_e.shape
    n_codes = emb.shape[0]
    z = jnp.transpose(z_e, (0, 2, 3, 1)).reshape(-1, c)
    zsq = jnp.sum(z * z, axis=1, keepdims=True)
    esq = jnp.sum(emb * emb, axis=1)[None, :]

    idx2, loss_sum = _argmin_call(z, emb, zsq, esq)
    idx = idx2.reshape(-1)

    st_flat = _make_sc_gather_st(n_codes, c, z.shape[0])(emb, idx, z)

    mean_sq = loss_sum[0, 0] / jnp.float32(z.size)
    loss = mean_sq + jnp.float32(_BETA) * mean_sq

    z_q_st = jnp.transpose(st_flat.reshape(b, h, w, c), (0, 3, 1, 2))
    return (z_q_st, loss, idx.reshape(b, h, w))


# speedup vs baseline: 1.3358x; 1.0953x over previous
import functools

import jax
import jax.numpy as jnp
from jax import lax
from jax.experimental import pallas as pl
from jax.experimental.pallas import tpu as pltpu
from jax.experimental.pallas import tpu_sc as plsc

_BETA = 0.25
_BM = 512
_BN = 2048
_INT_MAX = 2**31 - 1


def _argmin_body(z_ref, emb_ref, idx_ref, loss_ref, esq_scr, acc):
    i = pl.program_id(0)
    n_chunks = emb_ref.shape[0] // _BN

    @pl.when(i == 0)
    def _():
        e = emb_ref[...]
        esq_scr[...] = jnp.sum(e * e, axis=1)[None, :]

    z = z_ref[...]
    zm = z * -2.0
    zsq = jnp.sum(z * z, axis=1, keepdims=True)
    iota = lax.broadcasted_iota(jnp.int32, (_BM, _BN), 1).astype(jnp.float32)
    m_run = None
    a_run = None
    for c in range(n_chunks):
        mm = lax.dot_general(
            zm, emb_ref[pl.ds(c * _BN, _BN), :],
            dimension_numbers=(((1,), (1,)), ((), ())),
            preferred_element_type=jnp.float32,
        )
        dist = (zsq + mm) + esq_scr[:, pl.ds(c * _BN, _BN)]
        mloc = jnp.min(dist, axis=1, keepdims=True)
        aloc = jnp.min(jnp.where(dist == mloc, iota, jnp.inf),
                       axis=1, keepdims=True) + float(c * _BN)
        if c == 0:
            m_run, a_run = mloc, aloc
        else:
            better = mloc < m_run
            a_run = jnp.where(better, aloc, a_run)
            m_run = jnp.where(better, mloc, m_run)

    idx_ref[...] = a_run.astype(jnp.int32)
    part = jnp.sum(m_run)

    @pl.when(i == 0)
    def _():
        acc[0] = part

    @pl.when(i > 0)
    def _():
        acc[0] = acc[0] + part

    @pl.when(i == pl.num_programs(0) - 1)
    def _():
        loss_ref[0, 0] = acc[0]


def _argmin_call(z, emb):
    m, k = z.shape
    n = emb.shape[0]
    return pl.pallas_call(
        _argmin_body,
        grid=(m // _BM,),
        in_specs=[
            pl.BlockSpec((_BM, k), lambda i: (i, 0)),
            pl.BlockSpec((n, k), lambda i: (0, 0)),
        ],
        out_specs=[
            pl.BlockSpec((_BM, 1), lambda i: (i, 0)),
            pl.BlockSpec(memory_space=pltpu.SMEM),
        ],
        out_shape=[
            jax.ShapeDtypeStruct((m, 1), jnp.int32),
            jax.ShapeDtypeStruct((1, 1), jnp.float32),
        ],
        scratch_shapes=[
            pltpu.VMEM((1, n), jnp.float32),
            pltpu.SMEM((1,), jnp.float32),
        ],
        compiler_params=pltpu.CompilerParams(
            dimension_semantics=("arbitrary",),
        ),
    )(z, emb)


def _make_sc_gather_st(v, d, b):
    info = plsc.get_sparse_core_info()
    nw = info.num_cores * info.num_subcores
    b_per_w = b // nw
    lanes = info.num_lanes
    mesh = plsc.VectorSubcoreMesh(core_axis_name="c", subcore_axis_name="s")

    @functools.partial(
        pl.kernel, mesh=mesh,
        out_type=jax.ShapeDtypeStruct((b, d), jnp.float32),
        scratch_types=[
            pltpu.VMEM((b_per_w,), jnp.int32),
            pltpu.VMEM((b_per_w, d), jnp.float32),
            pltpu.VMEM((b_per_w, d), jnp.float32),
            pltpu.SemaphoreType.DMA,
        ],
    )
    def gather_st(table_hbm, idx_hbm, z_hbm, out_hbm, idx_v, rows_v, z_v, sem):
        wid = lax.axis_index("s") * info.num_cores + lax.axis_index("c")
        base = wid * b_per_w
        pltpu.sync_copy(idx_hbm.at[pl.ds(base, b_per_w)], idx_v)
        cp = pltpu.async_copy(table_hbm.at[idx_v], rows_v, sem)
        pltpu.sync_copy(z_hbm.at[pl.ds(base, b_per_w)], z_v)
        cp.wait()

        def row_body(r, carry):
            for j in range(d // lanes):
                zv = z_v[r, pl.ds(j * lanes, lanes)]
                ev = rows_v[r, pl.ds(j * lanes, lanes)]
                rows_v[r, pl.ds(j * lanes, lanes)] = zv + (ev - zv)
            return carry

        lax.fori_loop(0, b_per_w, row_body, 0)
        pltpu.sync_copy(rows_v, out_hbm.at[pl.ds(base, b_per_w)])

    return gather_st


def kernel(z_e, emb):
    b, c, h, w = z_e.shape
    n_codes = emb.shape[0]
    z = jnp.transpose(z_e, (0, 2, 3, 1)).reshape(-1, c)

    idx2, loss_sum = _argmin_call(z, emb)
    idx = idx2.reshape(-1)

    st_flat = _make_sc_gather_st(n_codes, c, z.shape[0])(emb, idx, z)

    mean_sq = loss_sum[0, 0] / jnp.float32(z.size)
    loss = mean_sq + jnp.float32(_BETA) * mean_sq

    z_q_st = jnp.transpose(st_flat.reshape(b, h, w, c), (0, 3, 1, 2))
    return (z_q_st, loss, idx.reshape(b, h, w))


# speedup vs baseline: 1.3385x; 1.0020x over previous
import functools

import jax
import jax.numpy as jnp
from jax import lax
from jax.experimental import pallas as pl
from jax.experimental.pallas import tpu as pltpu
from jax.experimental.pallas import tpu_sc as plsc

_BETA = 0.25
_BM = 512
_BN = 2048
_INT_MAX = 2**31 - 1


def _argmin_body(z_ref, emb_ref, idx_ref, loss_ref, esq_scr, acc):
    i = pl.program_id(0)
    n_chunks = emb_ref.shape[0] // _BN

    @pl.when(i == 0)
    def _():
        e = emb_ref[...]
        esq_scr[...] = jnp.sum(e * e, axis=1)[None, :]

    z = z_ref[...]
    zm = z * -2.0
    zsq = jnp.sum(z * z, axis=1, keepdims=True)
    iota = lax.broadcasted_iota(jnp.int32, (1, _BN), 1).astype(jnp.float32)
    m_run = None
    a_run = None
    for c in range(n_chunks):
        mm = lax.dot_general(
            zm, emb_ref[pl.ds(c * _BN, _BN), :],
            dimension_numbers=(((1,), (1,)), ((), ())),
            preferred_element_type=jnp.float32,
        )
        dist = (zsq + mm) + esq_scr[:, pl.ds(c * _BN, _BN)]
        mloc = jnp.min(dist, axis=1, keepdims=True)
        aloc = jnp.min(jnp.where(dist == mloc, iota, jnp.inf),
                       axis=1, keepdims=True) + float(c * _BN)
        if c == 0:
            m_run, a_run = mloc, aloc
        else:
            better = mloc < m_run
            a_run = jnp.where(better, aloc, a_run)
            m_run = jnp.where(better, mloc, m_run)

    idx_ref[...] = a_run.astype(jnp.int32)
    part = jnp.sum(m_run)

    @pl.when(i == 0)
    def _():
        acc[0] = part

    @pl.when(i > 0)
    def _():
        acc[0] = acc[0] + part

    @pl.when(i == pl.num_programs(0) - 1)
    def _():
        loss_ref[0, 0] = acc[0]


def _argmin_call(z, emb):
    m, k = z.shape
    n = emb.shape[0]
    return pl.pallas_call(
        _argmin_body,
        grid=(m // _BM,),
        in_specs=[
            pl.BlockSpec((_BM, k), lambda i: (i, 0)),
            pl.BlockSpec((n, k), lambda i: (0, 0)),
        ],
        out_specs=[
            pl.BlockSpec((_BM, 1), lambda i: (i, 0)),
            pl.BlockSpec(memory_space=pltpu.SMEM),
        ],
        out_shape=[
            jax.ShapeDtypeStruct((m, 1), jnp.int32),
            jax.ShapeDtypeStruct((1, 1), jnp.float32),
        ],
        scratch_shapes=[
            pltpu.VMEM((1, n), jnp.float32),
            pltpu.SMEM((1,), jnp.float32),
        ],
        compiler_params=pltpu.CompilerParams(
            dimension_semantics=("arbitrary",),
        ),
    )(z, emb)


def _make_sc_gather_st(v, d, b):
    info = plsc.get_sparse_core_info()
    nw = info.num_cores * info.num_subcores
    b_per_w = b // nw
    lanes = info.num_lanes
    mesh = plsc.VectorSubcoreMesh(core_axis_name="c", subcore_axis_name="s")

    @functools.partial(
        pl.kernel, mesh=mesh,
        out_type=jax.ShapeDtypeStruct((b, d), jnp.float32),
        scratch_types=[
            pltpu.VMEM((b_per_w,), jnp.int32),
            pltpu.VMEM((b_per_w, d), jnp.float32),
            pltpu.VMEM((b_per_w, d), jnp.float32),
            pltpu.SemaphoreType.DMA,
        ],
    )
    def gather_st(table_hbm, idx_hbm, z_hbm, out_hbm, idx_v, rows_v, z_v, sem):
        wid = lax.axis_index("s") * info.num_cores + lax.axis_index("c")
        base = wid * b_per_w
        pltpu.sync_copy(idx_hbm.at[pl.ds(base, b_per_w)], idx_v)
        cp = pltpu.async_copy(table_hbm.at[idx_v], rows_v, sem)
        pltpu.sync_copy(z_hbm.at[pl.ds(base, b_per_w)], z_v)
        cp.wait()

        def row_body(r, carry):
            for j in range(d // lanes):
                zv = z_v[r, pl.ds(j * lanes, lanes)]
                ev = rows_v[r, pl.ds(j * lanes, lanes)]
                rows_v[r, pl.ds(j * lanes, lanes)] = zv + (ev - zv)
            return carry

        lax.fori_loop(0, b_per_w, row_body, 0)
        pltpu.sync_copy(rows_v, out_hbm.at[pl.ds(base, b_per_w)])

    return gather_st


def kernel(z_e, emb):
    b, c, h, w = z_e.shape
    n_codes = emb.shape[0]
    z = jnp.transpose(z_e, (0, 2, 3, 1)).reshape(-1, c)

    idx2, loss_sum = _argmin_call(z, emb)
    idx = idx2.reshape(-1)

    st_flat = _make_sc_gather_st(n_codes, c, z.shape[0])(emb, idx, z)

    mean_sq = loss_sum[0, 0] / jnp.float32(z.size)
    loss = mean_sq + jnp.float32(_BETA) * mean_sq

    z_q_st = jnp.transpose(st_flat.reshape(b, h, w, c), (0, 3, 1, 2))
    return (z_q_st, loss, idx.reshape(b, h, w))


# speedup vs baseline: 1.3944x; 1.0417x over previous
import functools

import jax
import jax.numpy as jnp
from jax import lax
from jax.experimental import pallas as pl
from jax.experimental.pallas import tpu as pltpu
from jax.experimental.pallas import tpu_sc as plsc

_BETA = 0.25
_BM = 1024
_BN = 2048
_INT_MAX = 2**31 - 1


def _argmin_body(z_ref, emb_ref, idx_ref, loss_ref, esq_scr, acc):
    i = pl.program_id(0)
    n_chunks = emb_ref.shape[0] // _BN

    @pl.when(i == 0)
    def _():
        e = emb_ref[...]
        esq_scr[...] = jnp.sum(e * e, axis=1)[None, :]

    z = z_ref[...]
    zm = z * -2.0
    zsq = jnp.sum(z * z, axis=1, keepdims=True)
    iota = lax.broadcasted_iota(jnp.int32, (1, _BN), 1).astype(jnp.float32)
    m_run = None
    a_run = None
    for c in range(n_chunks):
        mm = lax.dot_general(
            zm, emb_ref[pl.ds(c * _BN, _BN), :],
            dimension_numbers=(((1,), (1,)), ((), ())),
            preferred_element_type=jnp.float32,
        )
        dist = (zsq + mm) + esq_scr[:, pl.ds(c * _BN, _BN)]
        mloc = jnp.min(dist, axis=1, keepdims=True)
        aloc = jnp.min(jnp.where(dist == mloc, iota, jnp.inf),
                       axis=1, keepdims=True) + float(c * _BN)
        if c == 0:
            m_run, a_run = mloc, aloc
        else:
            better = mloc < m_run
            a_run = jnp.where(better, aloc, a_run)
            m_run = jnp.where(better, mloc, m_run)

    idx_ref[...] = a_run.astype(jnp.int32)
    part = jnp.sum(m_run)

    @pl.when(i == 0)
    def _():
        acc[0] = part

    @pl.when(i > 0)
    def _():
        acc[0] = acc[0] + part

    @pl.when(i == pl.num_programs(0) - 1)
    def _():
        loss_ref[0, 0] = acc[0]


def _argmin_call(z, emb):
    m, k = z.shape
    n = emb.shape[0]
    return pl.pallas_call(
        _argmin_body,
        grid=(m // _BM,),
        in_specs=[
            pl.BlockSpec((_BM, k), lambda i: (i, 0)),
            pl.BlockSpec((n, k), lambda i: (0, 0)),
        ],
        out_specs=[
            pl.BlockSpec((_BM, 1), lambda i: (i, 0)),
            pl.BlockSpec(memory_space=pltpu.SMEM),
        ],
        out_shape=[
            jax.ShapeDtypeStruct((m, 1), jnp.int32),
            jax.ShapeDtypeStruct((1, 1), jnp.float32),
        ],
        scratch_shapes=[
            pltpu.VMEM((1, n), jnp.float32),
            pltpu.SMEM((1,), jnp.float32),
        ],
        compiler_params=pltpu.CompilerParams(
            dimension_semantics=("arbitrary",),
        ),
    )(z, emb)


def _make_sc_gather_st(v, d, b):
    info = plsc.get_sparse_core_info()
    nw = info.num_cores * info.num_subcores
    b_per_w = b // nw
    lanes = info.num_lanes
    mesh = plsc.VectorSubcoreMesh(core_axis_name="c", subcore_axis_name="s")

    @functools.partial(
        pl.kernel, mesh=mesh,
        out_type=jax.ShapeDtypeStruct((b, d), jnp.float32),
        scratch_types=[
            pltpu.VMEM((b_per_w,), jnp.int32),
            pltpu.VMEM((b_per_w, d), jnp.float32),
            pltpu.VMEM((b_per_w, d), jnp.float32),
            pltpu.SemaphoreType.DMA,
        ],
    )
    def gather_st(table_hbm, idx_hbm, z_hbm, out_hbm, idx_v, rows_v, z_v, sem):
        wid = lax.axis_index("s") * info.num_cores + lax.axis_index("c")
        base = wid * b_per_w
        pltpu.sync_copy(idx_hbm.at[pl.ds(base, b_per_w)], idx_v)
        cp = pltpu.async_copy(table_hbm.at[idx_v], rows_v, sem)
        pltpu.sync_copy(z_hbm.at[pl.ds(base, b_per_w)], z_v)
        cp.wait()

        def row_body(r, carry):
            for j in range(d // lanes):
                zv = z_v[r, pl.ds(j * lanes, lanes)]
                ev = rows_v[r, pl.ds(j * lanes, lanes)]
                rows_v[r, pl.ds(j * lanes, lanes)] = zv + (ev - zv)
            return carry

        lax.fori_loop(0, b_per_w, row_body, 0)
        pltpu.sync_copy(rows_v, out_hbm.at[pl.ds(base, b_per_w)])

    return gather_st


def kernel(z_e, emb):
    b, c, h, w = z_e.shape
    n_codes = emb.shape[0]
    z = jnp.transpose(z_e, (0, 2, 3, 1)).reshape(-1, c)

    idx2, loss_sum = _argmin_call(z, emb)
    idx = idx2.reshape(-1)

    st_flat = _make_sc_gather_st(n_codes, c, z.shape[0])(emb, idx, z)

    mean_sq = loss_sum[0, 0] / jnp.float32(z.size)
    loss = mean_sq + jnp.float32(_BETA) * mean_sq

    z_q_st = jnp.transpose(st_flat.reshape(b, h, w, c), (0, 3, 1, 2))
    return (z_q_st, loss, idx.reshape(b, h, w))


# speedup vs baseline: 1.4362x; 1.0299x over previous
import functools

import jax
import jax.numpy as jnp
from jax import lax
from jax.experimental import pallas as pl
from jax.experimental.pallas import tpu as pltpu
from jax.experimental.pallas import tpu_sc as plsc

_BETA = 0.25
_BM = 2048
_BN = 2048
_INT_MAX = 2**31 - 1


def _argmin_body(z_ref, emb_ref, idx_ref, loss_ref, esq_scr, acc):
    i = pl.program_id(0)
    n_chunks = emb_ref.shape[0] // _BN

    @pl.when(i == 0)
    def _():
        e = emb_ref[...]
        esq_scr[...] = jnp.sum(e * e, axis=1)[None, :]

    z = z_ref[...]
    zm = z * -2.0
    zsq = jnp.sum(z * z, axis=1, keepdims=True)
    iota = lax.broadcasted_iota(jnp.int32, (1, _BN), 1).astype(jnp.float32)
    m_run = None
    a_run = None
    for c in range(n_chunks):
        mm = lax.dot_general(
            zm, emb_ref[pl.ds(c * _BN, _BN), :],
            dimension_numbers=(((1,), (1,)), ((), ())),
            preferred_element_type=jnp.float32,
        )
        dist = (zsq + mm) + esq_scr[:, pl.ds(c * _BN, _BN)]
        mloc = jnp.min(dist, axis=1, keepdims=True)
        aloc = jnp.min(jnp.where(dist == mloc, iota, jnp.inf),
                       axis=1, keepdims=True) + float(c * _BN)
        if c == 0:
            m_run, a_run = mloc, aloc
        else:
            better = mloc < m_run
            a_run = jnp.where(better, aloc, a_run)
            m_run = jnp.where(better, mloc, m_run)

    idx_ref[...] = a_run.astype(jnp.int32)
    part = jnp.sum(m_run)

    @pl.when(i == 0)
    def _():
        acc[0] = part

    @pl.when(i > 0)
    def _():
        acc[0] = acc[0] + part

    @pl.when(i == pl.num_programs(0) - 1)
    def _():
        loss_ref[0, 0] = acc[0]


def _argmin_call(z, emb):
    m, k = z.shape
    n = emb.shape[0]
    return pl.pallas_call(
        _argmin_body,
        grid=(m // _BM,),
        in_specs=[
            pl.BlockSpec((_BM, k), lambda i: (i, 0)),
            pl.BlockSpec((n, k), lambda i: (0, 0)),
        ],
        out_specs=[
            pl.BlockSpec((_BM, 1), lambda i: (i, 0)),
            pl.BlockSpec(memory_space=pltpu.SMEM),
        ],
        out_shape=[
            jax.ShapeDtypeStruct((m, 1), jnp.int32),
            jax.ShapeDtypeStruct((1, 1), jnp.float32),
        ],
        scratch_shapes=[
            pltpu.VMEM((1, n), jnp.float32),
            pltpu.SMEM((1,), jnp.float32),
        ],
        compiler_params=pltpu.CompilerParams(
            dimension_semantics=("arbitrary",),
        ),
    )(z, emb)


def _make_sc_gather_st(v, d, b):
    info = plsc.get_sparse_core_info()
    nw = info.num_cores * info.num_subcores
    b_per_w = b // nw
    lanes = info.num_lanes
    mesh = plsc.VectorSubcoreMesh(core_axis_name="c", subcore_axis_name="s")

    @functools.partial(
        pl.kernel, mesh=mesh,
        out_type=jax.ShapeDtypeStruct((b, d), jnp.float32),
        scratch_types=[
            pltpu.VMEM((b_per_w,), jnp.int32),
            pltpu.VMEM((b_per_w, d), jnp.float32),
            pltpu.VMEM((b_per_w, d), jnp.float32),
            pltpu.SemaphoreType.DMA,
        ],
    )
    def gather_st(table_hbm, idx_hbm, z_hbm, out_hbm, idx_v, rows_v, z_v, sem):
        wid = lax.axis_index("s") * info.num_cores + lax.axis_index("c")
        base = wid * b_per_w
        pltpu.sync_copy(idx_hbm.at[pl.ds(base, b_per_w)], idx_v)
        cp = pltpu.async_copy(table_hbm.at[idx_v], rows_v, sem)
        pltpu.sync_copy(z_hbm.at[pl.ds(base, b_per_w)], z_v)
        cp.wait()

        def row_body(r, carry):
            for j in range(d // lanes):
                zv = z_v[r, pl.ds(j * lanes, lanes)]
                ev = rows_v[r, pl.ds(j * lanes, lanes)]
                rows_v[r, pl.ds(j * lanes, lanes)] = zv + (ev - zv)
            return carry

        lax.fori_loop(0, b_per_w, row_body, 0)
        pltpu.sync_copy(rows_v, out_hbm.at[pl.ds(base, b_per_w)])

    return gather_st


def kernel(z_e, emb):
    b, c, h, w = z_e.shape
    n_codes = emb.shape[0]
    z = jnp.transpose(z_e, (0, 2, 3, 1)).reshape(-1, c)

    idx2, loss_sum = _argmin_call(z, emb)
    idx = idx2.reshape(-1)

    st_flat = _make_sc_gather_st(n_codes, c, z.shape[0])(emb, idx, z)

    mean_sq = loss_sum[0, 0] / jnp.float32(z.size)
    loss = mean_sq + jnp.float32(_BETA) * mean_sq

    z_q_st = jnp.transpose(st_flat.reshape(b, h, w, c), (0, 3, 1, 2))
    return (z_q_st, loss, idx.reshape(b, h, w))
